# Initial kernel scaffold; baseline (speedup 1.0000x reference)
import jax
import jax.numpy as jnp
from jax.experimental import pallas as pl


def kernel(x, edge_index, W, b):
    raise NotImplementedError("write your pallas kernel here")


# speedup vs baseline: 9.6592x; 9.6592x over previous
import functools

import jax
import jax.numpy as jnp
from jax import lax
from jax.experimental import pallas as pl
from jax.experimental.pallas import tpu as pltpu
from jax.experimental.pallas import tpu_sc as plsc

N = 10000
E = 320000
C = 128
NC = 2
NS = 16
NW = NC * NS
CH = 128
NCHUNK = -(-E // (NW * CH))
ET = NCHUNK * CH
PAD_E = ET * NW
NPAD = 10240
NROWS_T = NPAD // NS

@functools.cache
def _sc_degree_kernel():
    mesh = plsc.VectorSubcoreMesh(core_axis_name="c", subcore_axis_name="s",
                                  num_cores=NC, num_subcores=NS)
    return pl.kernel(
        _sc_degree_body,
        out_type=jax.ShapeDtypeStruct((NC, NPAD, 16), jnp.float32),
        mesh=mesh,
        scratch_types=[
            pltpu.VMEM_SHARED((NPAD, 16), jnp.float32),
            pltpu.VMEM((CH,), jnp.int32),
            pltpu.VMEM((CH, 16), jnp.float32),
        ],
    )


def _sc_degree_body(col_hbm, out_hbm, acc, idx_v, ones_v):
    c = lax.axis_index("c")
    s = lax.axis_index("s")

    @pl.loop(0, CH)
    def _(i):
        ones_v[i, :] = jnp.zeros((16,), jnp.float32)

    @pl.loop(0, NROWS_T // CH)
    def _(j):
        pltpu.sync_copy(ones_v, acc.at[pl.ds(s * NROWS_T + j * CH, CH)])

    @pl.loop(0, CH)
    def _(i):
        ones_v[i, :] = jnp.full((16,), 1.0, jnp.float32)

    plsc.subcore_barrier()

    base = c * (PAD_E // NC) + s * ET

    @pl.loop(0, NCHUNK)
    def _(t):
        pltpu.sync_copy(col_hbm.at[pl.ds(base + t * CH, CH)], idx_v)
        pltpu.sync_copy(ones_v, acc.at[idx_v], add=True)

    plsc.subcore_barrier()
    pltpu.sync_copy(acc.at[pl.ds(s * NROWS_T, NROWS_T)],
                    out_hbm.at[c, pl.ds(s * NROWS_T, NROWS_T)])


@functools.cache
def _sc_prop_kernel():
    mesh = plsc.VectorSubcoreMesh(core_axis_name="c", subcore_axis_name="s",
                                  num_cores=NC, num_subcores=NS)
    return pl.kernel(
        _sc_prop_body,
        out_type=jax.ShapeDtypeStruct((NC, NPAD, C), jnp.float32),
        mesh=mesh,
        scratch_types=[
            pltpu.VMEM_SHARED((NPAD, C), jnp.float32),
            pltpu.VMEM((CH,), jnp.int32),
            pltpu.VMEM((CH,), jnp.int32),
            pltpu.VMEM((CH, C), jnp.float32),
            pltpu.VMEM((CH, C), jnp.float32),
        ],
    )


def _sc_prop_body(u_hbm, row_hbm, col_hbm, out_hbm, acc, idxr_v, idxc_v, rows_v, zero_v):
    c = lax.axis_index("c")
    s = lax.axis_index("s")

    @pl.loop(0, CH)
    def _(i):
        @pl.loop(0, C // 16)
        def _(j):
            zero_v[i, pl.ds(j * 16, 16)] = jnp.zeros((16,), jnp.float32)

    @pl.loop(0, NROWS_T // CH)
    def _(j):
        pltpu.sync_copy(zero_v, acc.at[pl.ds(s * NROWS_T + j * CH, CH)])

    plsc.subcore_barrier()

    base = c * (PAD_E // NC) + s * ET

    @pl.loop(0, NCHUNK)
    def _(t):
        pltpu.sync_copy(row_hbm.at[pl.ds(base + t * CH, CH)], idxr_v)
        pltpu.sync_copy(u_hbm.at[idxr_v], rows_v)
        pltpu.sync_copy(col_hbm.at[pl.ds(base + t * CH, CH)], idxc_v)
        pltpu.sync_copy(rows_v, acc.at[idxc_v], add=True)

    plsc.subcore_barrier()
    pltpu.sync_copy(acc.at[pl.ds(s * NROWS_T, NROWS_T)],
                    out_hbm.at[c, pl.ds(s * NROWS_T, NROWS_T)])


_BR = 256


def _tc_prep_body(dp_ref, x_ref, u0_ref, dinv_ref):
    deg = dp_ref[0, :, 0:1] + dp_ref[1, :, 0:1] + 1.0
    dinv = lax.rsqrt(deg)
    dinv_b = jnp.broadcast_to(dinv, (_BR, C))
    u0_ref[...] = dinv_b * x_ref[...]
    dinv_ref[...] = dinv_b


def _tc_prep(dp, x_pad):
    return pl.pallas_call(
        _tc_prep_body,
        grid=(NPAD // _BR,),
        in_specs=[
            pl.BlockSpec((NC, _BR, 16), lambda i: (0, i, 0)),
            pl.BlockSpec((_BR, C), lambda i: (i, 0)),
        ],
        out_specs=[
            pl.BlockSpec((_BR, C), lambda i: (i, 0)),
            pl.BlockSpec((_BR, C), lambda i: (i, 0)),
        ],
        out_shape=[
            jax.ShapeDtypeStruct((NPAD, C), jnp.float32),
            jax.ShapeDtypeStruct((NPAD, C), jnp.float32),
        ],
    )(dp, x_pad)


def _tc_mid_body(sp_ref, u_ref, dv_ref, o_ref):
    i = pl.program_id(0)
    t = sp_ref[0] + sp_ref[1] + u_ref[...]
    dv = dv_ref[...]
    rows = lax.broadcasted_iota(jnp.int32, (_BR, C), 0) + i * _BR
    o_ref[...] = jnp.where(rows < N, dv * dv * t, 0.0)


def _tc_mid(sp, u0, dinv_b):
    return pl.pallas_call(
        _tc_mid_body,
        grid=(NPAD // _BR,),
        in_specs=[
            pl.BlockSpec((NC, _BR, C), lambda i: (0, i, 0)),
            pl.BlockSpec((_BR, C), lambda i: (i, 0)),
            pl.BlockSpec((_BR, C), lambda i: (i, 0)),
        ],
        out_specs=pl.BlockSpec((_BR, C), lambda i: (i, 0)),
        out_shape=jax.ShapeDtypeStruct((NPAD, C), jnp.float32),
    )(sp, u0, dinv_b)


_BR2 = 200


def _tc_final_body(sp_ref, u_ref, dv_ref, w_ref, b_ref, o_ref):
    h2 = dv_ref[...] * (sp_ref[0] + sp_ref[1] + u_ref[...])
    z = jnp.dot(h2, w_ref[...], preferred_element_type=jnp.float32) + b_ref[...]
    m = jnp.max(z, axis=-1, keepdims=True)
    e = jnp.exp(z - m)
    o_ref[...] = (z - m) - jnp.log(jnp.sum(e, axis=-1, keepdims=True))


def _tc_final(sp, u1, dinv_b, W, b2):
    return pl.pallas_call(
        _tc_final_body,
        grid=(N // _BR2,),
        in_specs=[
            pl.BlockSpec((NC, _BR2, C), lambda i: (0, i, 0)),
            pl.BlockSpec((_BR2, C), lambda i: (i, 0)),
            pl.BlockSpec((_BR2, C), lambda i: (i, 0)),
            pl.BlockSpec((C, C), lambda i: (0, 0)),
            pl.BlockSpec((1, C), lambda i: (0, 0)),
        ],
        out_specs=pl.BlockSpec((_BR2, C), lambda i: (i, 0)),
        out_shape=jax.ShapeDtypeStruct((N, C), jnp.float32),
    )(sp, u1, dinv_b, W, b2)


def kernel(x, edge_index, W, b):
    pad = jnp.full((PAD_E - E,), N, dtype=jnp.int32)
    rowp = jnp.concatenate([edge_index[0], pad])
    colp = jnp.concatenate([edge_index[1], pad])
    x_pad = jnp.pad(x, ((0, NPAD - N), (0, 0)))

    dp = _sc_degree_kernel()(colp)
    u0, dinv_b = _tc_prep(dp, x_pad)
    s0 = _sc_prop_kernel()(u0, rowp, colp)
    u1 = _tc_mid(s0, u0, dinv_b)
    s1 = _sc_prop_kernel()(u1, rowp, colp)
    return _tc_final(s1, u1, dinv_b, W, b.reshape(1, C))
